# Initial kernel scaffold; baseline (speedup 1.0000x reference)
import functools

import jax
import jax.numpy as jnp
from jax import lax
from jax.experimental import pallas as pl
from jax.experimental.pallas import tpu as pltpu
from jax.experimental.pallas import tpu_sc as plsc

N = 10000
E = 320000
CH = 128
NC = 2
NS = 16
NW = NC * NS
EPW = E // NW
CHUNK = 128
NFULL = EPW // CHUNK
TAIL = EPW - NFULL * CHUNK
RPS = N // NS
DEG_PAD = 10240
DEG_STRIPE = DEG_PAD // NS

_mesh = plsc.VectorSubcoreMesh(core_axis_name="c", subcore_axis_name="s")


@functools.partial(
    pl.kernel,
    out_type=jax.ShapeDtypeStruct((NC * DEG_PAD,), jnp.float32),
    mesh=_mesh,
    scratch_types=[
        pltpu.VMEM((CHUNK,), jnp.int32),
        pltpu.VMEM((TAIL,), jnp.int32),
        pltpu.VMEM((CHUNK,), jnp.float32),
        pltpu.VMEM((DEG_STRIPE,), jnp.float32),
        pltpu.VMEM_SHARED((DEG_PAD,), jnp.float32),
    ],
)
def _deg(col_hbm, out_hbm, idx_v, idxt_v, ones_v, z_v, acc):
    c = lax.axis_index("c")
    s = lax.axis_index("s")
    wid = c * NS + s
    one16 = jnp.full((16,), 1.0, jnp.float32)
    zero16 = jnp.zeros((16,), jnp.float32)
    for j in range(CHUNK // 16):
        ones_v[pl.ds(j * 16, 16)] = one16
    for j in range(DEG_STRIPE // 16):
        z_v[pl.ds(j * 16, 16)] = zero16
    pltpu.sync_copy(z_v, acc.at[pl.ds(s * DEG_STRIPE, DEG_STRIPE)])
    plsc.subcore_barrier()

    @pl.loop(0, NFULL)
    def _(i):
        base = wid * EPW + i * CHUNK
        pltpu.sync_copy(col_hbm.at[pl.ds(base, CHUNK)], idx_v)
        pltpu.sync_copy(ones_v, acc.at[idx_v], add=True)

    tbase = wid * EPW + NFULL * CHUNK
    pltpu.sync_copy(col_hbm.at[pl.ds(tbase, TAIL)], idxt_v)
    pltpu.sync_copy(ones_v.at[pl.ds(0, TAIL)], acc.at[idxt_v], add=True)
    plsc.subcore_barrier()
    pltpu.sync_copy(
        acc.at[pl.ds(s * DEG_STRIPE, DEG_STRIPE)],
        out_hbm.at[pl.ds(c * DEG_PAD + s * DEG_STRIPE, DEG_STRIPE)],
    )


@functools.partial(
    pl.kernel,
    out_type=jax.ShapeDtypeStruct((NC * N, CH), jnp.float32),
    mesh=_mesh,
    scratch_types=[
        pltpu.VMEM((CHUNK,), jnp.int32),
        pltpu.VMEM((CHUNK,), jnp.int32),
        pltpu.VMEM((TAIL,), jnp.int32),
        pltpu.VMEM((TAIL,), jnp.int32),
        pltpu.VMEM((CHUNK, CH), jnp.float32),
        pltpu.VMEM((TAIL, CH), jnp.float32),
        pltpu.VMEM_SHARED((N, CH), jnp.float32),
        pltpu.SemaphoreType.DMA,
    ],
)
def _prop(g_hbm, row_hbm, col_hbm, z_hbm, out_hbm,
          ridx, cidx, ridxt, cidxt, rows, rowst, acc, sem):
    c = lax.axis_index("c")
    s = lax.axis_index("s")
    wid = c * NS + s
    pltpu.sync_copy(z_hbm, acc.at[pl.ds(s * RPS, RPS)])
    plsc.subcore_barrier()

    @pl.loop(0, NFULL)
    def _(i):
        base = wid * EPW + i * CHUNK
        pltpu.sync_copy(row_hbm.at[pl.ds(base, CHUNK)], ridx)
        pltpu.sync_copy(col_hbm.at[pl.ds(base, CHUNK)], cidx)
        pltpu.async_copy(g_hbm.at[ridx], rows, sem).wait()
        pltpu.sync_copy(rows, acc.at[cidx], add=True)

    tbase = wid * EPW + NFULL * CHUNK
    pltpu.sync_copy(row_hbm.at[pl.ds(tbase, TAIL)], ridxt)
    pltpu.sync_copy(col_hbm.at[pl.ds(tbase, TAIL)], cidxt)
    pltpu.async_copy(g_hbm.at[ridxt], rowst, sem).wait()
    pltpu.sync_copy(rowst, acc.at[cidxt], add=True)
    plsc.subcore_barrier()
    pltpu.sync_copy(
        acc.at[pl.ds(s * RPS, RPS)],
        out_hbm.at[pl.ds(c * N + s * RPS, RPS)],
    )


_BLK = 1000
_GRID = N // _BLK

_row_spec = pl.BlockSpec((_BLK, CH), lambda i: (i, 0))
_col_spec = pl.BlockSpec((_BLK, 1), lambda i: (i, 0))
_w_spec = pl.BlockSpec((CH, CH), lambda i: (0, 0))
_b_spec = pl.BlockSpec((1, CH), lambda i: (0, 0))


def _t0_body(x_r, w0_r, da_r, db_r, out_r, g_r, dinv_r):
    d = da_r[...] + db_r[...]
    dinv = jnp.where(d > 0, lax.rsqrt(jnp.maximum(d, 1e-12)), 0.0)
    x = x_r[...]
    out_r[...] = jnp.dot(x, w0_r[...], preferred_element_type=jnp.float32)
    g_r[...] = x * dinv
    dinv_r[...] = dinv


_t0 = pl.pallas_call(
    _t0_body,
    grid=(_GRID,),
    in_specs=[_row_spec, _w_spec, _col_spec, _col_spec],
    out_specs=[_row_spec, _row_spec, _col_spec],
    out_shape=[
        jax.ShapeDtypeStruct((N, CH), jnp.float32),
        jax.ShapeDtypeStruct((N, CH), jnp.float32),
        jax.ShapeDtypeStruct((N, 1), jnp.float32),
    ],
)


def _tmid_body(pa_r, pb_r, dinv_r, w_r, acc_r, out_r, g_r):
    dinv = dinv_r[...]
    q = (pa_r[...] + pb_r[...]) * dinv
    out_r[...] = acc_r[...] + jnp.dot(q, w_r[...], preferred_element_type=jnp.float32)
    g_r[...] = q * dinv


_tmid = pl.pallas_call(
    _tmid_body,
    grid=(_GRID,),
    in_specs=[_row_spec, _row_spec, _col_spec, _w_spec, _row_spec],
    out_specs=[_row_spec, _row_spec],
    out_shape=[
        jax.ShapeDtypeStruct((N, CH), jnp.float32),
        jax.ShapeDtypeStruct((N, CH), jnp.float32),
    ],
)


def _ttrans_body(pa_r, pb_r, dinv_r, w13_r, acc_r, b1_r, w20_r, b2_r, out_r, g_r):
    dinv = dinv_r[...]
    q = (pa_r[...] + pb_r[...]) * dinv
    h = acc_r[...] + jnp.dot(q, w13_r[...], preferred_element_type=jnp.float32) + b1_r[...]
    h = jnp.maximum(h, 0.0)
    out_r[...] = jnp.dot(h, w20_r[...], preferred_element_type=jnp.float32) + b2_r[...]
    g_r[...] = h * dinv


_ttrans = pl.pallas_call(
    _ttrans_body,
    grid=(_GRID,),
    in_specs=[_row_spec, _row_spec, _col_spec, _w_spec, _row_spec, _b_spec,
              _w_spec, _b_spec],
    out_specs=[_row_spec, _row_spec],
    out_shape=[
        jax.ShapeDtypeStruct((N, CH), jnp.float32),
        jax.ShapeDtypeStruct((N, CH), jnp.float32),
    ],
)


def _tfinal_body(pa_r, pb_r, dinv_r, w_r, acc_r, out_r):
    q = (pa_r[...] + pb_r[...]) * dinv_r[...]
    o = acc_r[...] + jnp.dot(q, w_r[...], preferred_element_type=jnp.float32)
    m = jnp.max(o, axis=1, keepdims=True)
    lse = jnp.log(jnp.sum(jnp.exp(o - m), axis=1, keepdims=True)) + m
    out_r[...] = o - lse


_tfinal = pl.pallas_call(
    _tfinal_body,
    grid=(_GRID,),
    in_specs=[_row_spec, _row_spec, _col_spec, _w_spec, _row_spec],
    out_specs=_row_spec,
    out_shape=jax.ShapeDtypeStruct((N, CH), jnp.float32),
)


def kernel(x, edge_index, W1_0, W1_1, W1_2, W1_3, b1, W2_0, W2_1, W2_2, W2_3, b2):
    row = edge_index[0].astype(jnp.int32)
    col = edge_index[1].astype(jnp.int32)
    zrows = jnp.zeros((RPS, CH), jnp.float32)
    b1r = b1.reshape(1, CH)
    b2r = b2.reshape(1, CH)

    degp = _deg(col)
    da = degp[:N].reshape(N, 1)
    db = degp[DEG_PAD:DEG_PAD + N].reshape(N, 1)

    out, g, dinv = _t0(x, W1_0, da, db)

    for W in (W1_1, W1_2):
        p = _prop(g, row, col, zrows)
        out, g = _tmid(p[:N], p[N:], dinv, W, out)

    p = _prop(g, row, col, zrows)
    out, g = _ttrans(p[:N], p[N:], dinv, W1_3, out, b1r, W2_0, b2r)

    for W in (W2_1, W2_2):
        p = _prop(g, row, col, zrows)
        out, g = _tmid(p[:N], p[N:], dinv, W, out)

    p = _prop(g, row, col, zrows)
    return _tfinal(p[:N], p[N:], dinv, W2_3, out)


# speedup vs baseline: 8.3707x; 8.3707x over previous
import functools

import jax
import jax.numpy as jnp
from jax import lax
from jax.experimental import pallas as pl
from jax.experimental.pallas import tpu as pltpu
from jax.experimental.pallas import tpu_sc as plsc

N = 10000
E = 320000
CH = 128
NC = 2
NS = 16
NW = NC * NS
EPW = E // NW
CHUNK = 128
NFULL = EPW // CHUNK
TAIL = EPW - NFULL * CHUNK
RPS = 632
N_PAD = NS * RPS
DEG_PAD = 10240
DEG_STRIPE = DEG_PAD // NS

_mesh = plsc.VectorSubcoreMesh(core_axis_name="c", subcore_axis_name="s")


@functools.partial(
    pl.kernel,
    out_type=jax.ShapeDtypeStruct((NC * DEG_PAD,), jnp.float32),
    mesh=_mesh,
    scratch_types=[
        pltpu.VMEM((CHUNK,), jnp.int32),
        pltpu.VMEM((TAIL,), jnp.int32),
        pltpu.VMEM((CHUNK,), jnp.float32),
        pltpu.VMEM((DEG_STRIPE,), jnp.float32),
        pltpu.VMEM_SHARED((DEG_PAD,), jnp.float32),
    ],
)
def _deg(col_hbm, out_hbm, idx_v, idxt_v, ones_v, z_v, acc):
    c = lax.axis_index("c")
    s = lax.axis_index("s")
    wid = c * NS + s
    one16 = jnp.full((16,), 1.0, jnp.float32)
    zero16 = jnp.zeros((16,), jnp.float32)
    for j in range(CHUNK // 16):
        ones_v[pl.ds(j * 16, 16)] = one16
    for j in range(DEG_STRIPE // 16):
        z_v[pl.ds(j * 16, 16)] = zero16
    pltpu.sync_copy(z_v, acc.at[pl.ds(s * DEG_STRIPE, DEG_STRIPE)])
    plsc.subcore_barrier()

    @pl.loop(0, NFULL)
    def _(i):
        base = wid * EPW + i * CHUNK
        pltpu.sync_copy(col_hbm.at[pl.ds(base, CHUNK)], idx_v)
        pltpu.sync_copy(ones_v, acc.at[idx_v], add=True)

    tbase = wid * EPW + NFULL * CHUNK
    pltpu.sync_copy(col_hbm.at[pl.ds(tbase, TAIL)], idxt_v)
    pltpu.sync_copy(ones_v.at[pl.ds(0, TAIL)], acc.at[idxt_v], add=True)
    plsc.subcore_barrier()
    pltpu.sync_copy(
        acc.at[pl.ds(s * DEG_STRIPE, DEG_STRIPE)],
        out_hbm.at[pl.ds(c * DEG_PAD + s * DEG_STRIPE, DEG_STRIPE)],
    )


@functools.partial(
    pl.kernel,
    out_type=jax.ShapeDtypeStruct((NC * N_PAD, CH), jnp.float32),
    mesh=_mesh,
    scratch_types=[
        pltpu.VMEM((CHUNK,), jnp.int32),
        pltpu.VMEM((CHUNK,), jnp.int32),
        pltpu.VMEM((TAIL,), jnp.int32),
        pltpu.VMEM((TAIL,), jnp.int32),
        pltpu.VMEM((CHUNK, CH), jnp.float32),
        pltpu.VMEM((TAIL, CH), jnp.float32),
        pltpu.VMEM_SHARED((N_PAD, CH), jnp.float32),
        pltpu.SemaphoreType.DMA,
    ],
)
def _prop(g_hbm, row_hbm, col_hbm, z_hbm, out_hbm,
          ridx, cidx, ridxt, cidxt, rows, rowst, acc, sem):
    c = lax.axis_index("c")
    s = lax.axis_index("s")
    wid = c * NS + s
    pltpu.sync_copy(z_hbm, acc.at[pl.ds(s * RPS, RPS)])
    plsc.subcore_barrier()

    @pl.loop(0, NFULL)
    def _(i):
        base = wid * EPW + i * CHUNK
        pltpu.sync_copy(row_hbm.at[pl.ds(base, CHUNK)], ridx)
        pltpu.sync_copy(col_hbm.at[pl.ds(base, CHUNK)], cidx)
        pltpu.async_copy(g_hbm.at[ridx], rows, sem).wait()
        pltpu.sync_copy(rows, acc.at[cidx], add=True)

    tbase = wid * EPW + NFULL * CHUNK
    pltpu.sync_copy(row_hbm.at[pl.ds(tbase, TAIL)], ridxt)
    pltpu.sync_copy(col_hbm.at[pl.ds(tbase, TAIL)], cidxt)
    pltpu.async_copy(g_hbm.at[ridxt], rowst, sem).wait()
    pltpu.sync_copy(rowst, acc.at[cidxt], add=True)
    plsc.subcore_barrier()
    pltpu.sync_copy(
        acc.at[pl.ds(s * RPS, RPS)],
        out_hbm.at[pl.ds(c * N_PAD + s * RPS, RPS)],
    )


_BLK = 1000
_GRID = N // _BLK

_row_spec = pl.BlockSpec((_BLK, CH), lambda i: (i, 0))
_col_spec = pl.BlockSpec((_BLK, 1), lambda i: (i, 0))
_w_spec = pl.BlockSpec((CH, CH), lambda i: (0, 0))
_b_spec = pl.BlockSpec((1, CH), lambda i: (0, 0))


def _t0_body(x_r, w0_r, da_r, db_r, out_r, g_r, dinv_r):
    d = da_r[...] + db_r[...]
    dinv = jnp.where(d > 0, lax.rsqrt(jnp.maximum(d, 1e-12)), 0.0)
    x = x_r[...]
    out_r[...] = jnp.dot(x, w0_r[...], preferred_element_type=jnp.float32)
    g_r[...] = x * dinv
    dinv_r[...] = dinv


_t0 = pl.pallas_call(
    _t0_body,
    grid=(_GRID,),
    in_specs=[_row_spec, _w_spec, _col_spec, _col_spec],
    out_specs=[_row_spec, _row_spec, _col_spec],
    out_shape=[
        jax.ShapeDtypeStruct((N, CH), jnp.float32),
        jax.ShapeDtypeStruct((N, CH), jnp.float32),
        jax.ShapeDtypeStruct((N, 1), jnp.float32),
    ],
)


def _tmid_body(pa_r, pb_r, dinv_r, w_r, acc_r, out_r, g_r):
    dinv = dinv_r[...]
    q = (pa_r[...] + pb_r[...]) * dinv
    out_r[...] = acc_r[...] + jnp.dot(q, w_r[...], preferred_element_type=jnp.float32)
    g_r[...] = q * dinv


_tmid = pl.pallas_call(
    _tmid_body,
    grid=(_GRID,),
    in_specs=[_row_spec, _row_spec, _col_spec, _w_spec, _row_spec],
    out_specs=[_row_spec, _row_spec],
    out_shape=[
        jax.ShapeDtypeStruct((N, CH), jnp.float32),
        jax.ShapeDtypeStruct((N, CH), jnp.float32),
    ],
)


def _ttrans_body(pa_r, pb_r, dinv_r, w13_r, acc_r, b1_r, w20_r, b2_r, out_r, g_r):
    dinv = dinv_r[...]
    q = (pa_r[...] + pb_r[...]) * dinv
    h = acc_r[...] + jnp.dot(q, w13_r[...], preferred_element_type=jnp.float32) + b1_r[...]
    h = jnp.maximum(h, 0.0)
    out_r[...] = jnp.dot(h, w20_r[...], preferred_element_type=jnp.float32) + b2_r[...]
    g_r[...] = h * dinv


_ttrans = pl.pallas_call(
    _ttrans_body,
    grid=(_GRID,),
    in_specs=[_row_spec, _row_spec, _col_spec, _w_spec, _row_spec, _b_spec,
              _w_spec, _b_spec],
    out_specs=[_row_spec, _row_spec],
    out_shape=[
        jax.ShapeDtypeStruct((N, CH), jnp.float32),
        jax.ShapeDtypeStruct((N, CH), jnp.float32),
    ],
)


def _tfinal_body(pa_r, pb_r, dinv_r, w_r, acc_r, out_r):
    q = (pa_r[...] + pb_r[...]) * dinv_r[...]
    o = acc_r[...] + jnp.dot(q, w_r[...], preferred_element_type=jnp.float32)
    m = jnp.max(o, axis=1, keepdims=True)
    lse = jnp.log(jnp.sum(jnp.exp(o - m), axis=1, keepdims=True)) + m
    out_r[...] = o - lse


_tfinal = pl.pallas_call(
    _tfinal_body,
    grid=(_GRID,),
    in_specs=[_row_spec, _row_spec, _col_spec, _w_spec, _row_spec],
    out_specs=_row_spec,
    out_shape=jax.ShapeDtypeStruct((N, CH), jnp.float32),
)


def kernel(x, edge_index, W1_0, W1_1, W1_2, W1_3, b1, W2_0, W2_1, W2_2, W2_3, b2):
    row = edge_index[0].astype(jnp.int32)
    col = edge_index[1].astype(jnp.int32)
    zrows = jnp.zeros((RPS, CH), jnp.float32)
    b1r = b1.reshape(1, CH)
    b2r = b2.reshape(1, CH)

    degp = _deg(col)
    da = degp[:N].reshape(N, 1)
    db = degp[DEG_PAD:DEG_PAD + N].reshape(N, 1)

    out, g, dinv = _t0(x, W1_0, da, db)

    for W in (W1_1, W1_2):
        p = _prop(g, row, col, zrows)
        out, g = _tmid(p[:N], p[N_PAD:N_PAD + N], dinv, W, out)

    p = _prop(g, row, col, zrows)
    out, g = _ttrans(p[:N], p[N_PAD:N_PAD + N], dinv, W1_3, out, b1r, W2_0, b2r)

    for W in (W2_1, W2_2):
        p = _prop(g, row, col, zrows)
        out, g = _tmid(p[:N], p[N_PAD:N_PAD + N], dinv, W, out)

    p = _prop(g, row, col, zrows)
    return _tfinal(p[:N], p[N_PAD:N_PAD + N], dinv, W2_3, out)
